# Initial kernel scaffold; baseline (speedup 1.0000x reference)
import jax
import jax.numpy as jnp
from jax.experimental import pallas as pl


def kernel(z, edge_index):
    raise NotImplementedError("write your pallas kernel here")


# speedup vs baseline: 1.0982x; 1.0982x over previous
import functools

import jax
import jax.numpy as jnp
from jax import lax
from jax.experimental import pallas as pl
from jax.experimental.pallas import tpu as pltpu
from jax.experimental.pallas import tpu_sc as plsc

D = 128
L = 16
NC = 2
NS = 16
NW = NC * NS
C = 80


@functools.lru_cache(maxsize=None)
def _make_sc_decoder(B: int):
    b_per_w = B // NW
    n_chunks = b_per_w // C
    mesh = plsc.VectorSubcoreMesh(core_axis_name="c", subcore_axis_name="s")

    @functools.partial(
        pl.kernel,
        mesh=mesh,
        out_type=jax.ShapeDtypeStruct((B,), jnp.float32),
        compiler_params=pltpu.CompilerParams(needs_layout_passes=False),
        scratch_types=[
            pltpu.VMEM((C,), jnp.int32),
            pltpu.VMEM((C,), jnp.int32),
            pltpu.VMEM((C, D), jnp.float32),
            pltpu.VMEM((C, D), jnp.float32),
            pltpu.VMEM((C,), jnp.float32),
            pltpu.SemaphoreType.DMA,
            pltpu.SemaphoreType.DMA,
        ],
    )
    def body(z_hbm, row_hbm, col_hbm, out_hbm,
             ridx_v, cidx_v, rows_v, cols_v, out_v, sem_r, sem_c):
        wid = lax.axis_index("s") * NC + lax.axis_index("c")
        base = wid * b_per_w

        def chunk_body(ci, carry):
            off = base + ci * C
            pltpu.sync_copy(row_hbm.at[pl.ds(off, C)], ridx_v)
            pltpu.sync_copy(col_hbm.at[pl.ds(off, C)], cidx_v)
            cp_r = pltpu.async_copy(z_hbm.at[ridx_v], rows_v, sem_r)
            cp_c = pltpu.async_copy(z_hbm.at[cidx_v], cols_v, sem_c)
            cp_r.wait()
            cp_c.wait()

            def group_body(g, c2):
                eb = g * L
                lanes = eb + lax.iota(jnp.int32, L)
                acc = jnp.zeros((L,), jnp.float32)
                for d in range(D):
                    dcol = jnp.full((L,), d, jnp.int32)
                    a = plsc.load_gather(rows_v, [lanes, dcol])
                    b = plsc.load_gather(cols_v, [lanes, dcol])
                    acc = acc + a * b
                out_v[pl.ds(eb, L)] = 1.0 / (1.0 + jnp.exp(-acc))
                return c2

            lax.fori_loop(0, C // L, group_body, 0)
            pltpu.sync_copy(out_v, out_hbm.at[pl.ds(off, C)])
            return carry

        lax.fori_loop(0, n_chunks, chunk_body, 0)

    return body


def kernel(z, edge_index):
    ei = edge_index.astype(jnp.int32)
    return _make_sc_decoder(ei.shape[1])(z, ei[0], ei[1])


# speedup vs baseline: 3.2122x; 2.9249x over previous
import functools

import jax
import jax.numpy as jnp
from jax import lax
from jax.experimental import pallas as pl
from jax.experimental.pallas import tpu as pltpu
from jax.experimental.pallas import tpu_sc as plsc

D = 128
L = 16
NC = 2
NS = 16
NW = NC * NS
C = 80


@functools.lru_cache(maxsize=None)
def _make_sc_decoder(B: int):
    b_per_w = B // NW
    n_chunks = b_per_w // C
    mesh = plsc.VectorSubcoreMesh(core_axis_name="c", subcore_axis_name="s")

    @functools.partial(
        pl.kernel,
        mesh=mesh,
        out_type=jax.ShapeDtypeStruct((B,), jnp.float32),
        compiler_params=pltpu.CompilerParams(needs_layout_passes=False),
        scratch_types=[
            pltpu.VMEM((b_per_w,), jnp.int32),
            pltpu.VMEM((b_per_w,), jnp.int32),
            pltpu.VMEM((C, D), jnp.float32),
            pltpu.VMEM((C, D), jnp.float32),
            pltpu.VMEM((C,), jnp.float32),
            pltpu.SemaphoreType.DMA,
            pltpu.SemaphoreType.DMA,
        ],
    )
    def body(z_hbm, row_hbm, col_hbm, out_hbm,
             ridx_v, cidx_v, rows_v, cols_v, out_v, sem_r, sem_c):
        wid = lax.axis_index("s") * NC + lax.axis_index("c")
        base = wid * b_per_w
        pltpu.sync_copy(row_hbm.at[pl.ds(base, b_per_w)], ridx_v)
        pltpu.sync_copy(col_hbm.at[pl.ds(base, b_per_w)], cidx_v)

        def chunk_body(ci, carry):
            coff = ci * C
            cp_r = pltpu.async_copy(
                z_hbm.at[ridx_v.at[pl.ds(coff, C)]], rows_v, sem_r)
            cp_c = pltpu.async_copy(
                z_hbm.at[cidx_v.at[pl.ds(coff, C)]], cols_v, sem_c)
            cp_r.wait()
            cp_c.wait()

            iota = lax.iota(jnp.int32, L)

            def group_body(g, c2):
                eb = g * L
                lanes = eb + iota
                acc = jnp.zeros((L,), jnp.float32)
                for t in range(D):
                    dcol = (iota + t) & (D - 1)
                    a = plsc.load_gather(rows_v, [lanes, dcol])
                    b = plsc.load_gather(cols_v, [lanes, dcol])
                    acc = acc + a * b
                out_v[pl.ds(eb, L)] = 1.0 / (1.0 + jnp.exp(-acc))
                return c2

            lax.fori_loop(0, C // L, group_body, 0)
            pltpu.sync_copy(out_v, out_hbm.at[pl.ds(base + coff, C)])
            return carry

        lax.fori_loop(0, n_chunks, chunk_body, 0)

    return body


def kernel(z, edge_index):
    ei = edge_index.astype(jnp.int32)
    return _make_sc_decoder(ei.shape[1])(z, ei[0], ei[1])


# speedup vs baseline: 11.0450x; 3.4384x over previous
import functools

import jax
import jax.numpy as jnp
from jax import lax
from jax.experimental import pallas as pl
from jax.experimental.pallas import tpu as pltpu
from jax.experimental.pallas import tpu_sc as plsc

D = 128
L = 16
NC = 2
NS = 16
NW = NC * NS
C = 80
NBUF = 4


@functools.lru_cache(maxsize=None)
def _make_sc_decoder(B: int):
    b_per_w = B // NW
    n_chunks = b_per_w // C
    mesh = plsc.VectorSubcoreMesh(core_axis_name="c", subcore_axis_name="s")

    @functools.partial(
        pl.kernel,
        mesh=mesh,
        out_type=jax.ShapeDtypeStruct((B,), jnp.float32),
        compiler_params=pltpu.CompilerParams(needs_layout_passes=False),
        scratch_types=[
            pltpu.VMEM((b_per_w,), jnp.int32),
            pltpu.VMEM((b_per_w,), jnp.int32),
            [pltpu.VMEM((C, D), jnp.float32) for _ in range(NBUF)],
            [pltpu.VMEM((C, D), jnp.float32) for _ in range(NBUF)],
            pltpu.VMEM((C,), jnp.float32),
            [pltpu.SemaphoreType.DMA for _ in range(NBUF)],
            [pltpu.SemaphoreType.DMA for _ in range(NBUF)],
        ],
    )
    def body(z_hbm, row_hbm, col_hbm, out_hbm,
             ridx_v, cidx_v, rows_bufs, cols_bufs, out_v, sems_r, sems_c):
        wid = lax.axis_index("s") * NC + lax.axis_index("c")
        base = wid * b_per_w
        pltpu.sync_copy(row_hbm.at[pl.ds(base, b_per_w)], ridx_v)
        pltpu.sync_copy(col_hbm.at[pl.ds(base, b_per_w)], cidx_v)

        def launch(ci, b):
            coff = ci * C
            pltpu.async_copy(
                z_hbm.at[ridx_v.at[pl.ds(coff, C)]], rows_bufs[b], sems_r[b])
            pltpu.async_copy(
                z_hbm.at[cidx_v.at[pl.ds(coff, C)]], cols_bufs[b], sems_c[b])

        for b in range(NBUF):
            launch(b, b)

        iota = lax.iota(jnp.int32, L)

        def compute(ci, b):
            rows_v, cols_v = rows_bufs[b], cols_bufs[b]
            pltpu.make_async_copy(z_hbm.at[ridx_v.at[pl.ds(0, C)]],
                                  rows_v, sems_r[b]).wait()
            pltpu.make_async_copy(z_hbm.at[cidx_v.at[pl.ds(0, C)]],
                                  cols_v, sems_c[b]).wait()

            def group_body(g, c2):
                eb = g * L
                lanes = eb + iota
                def t_body(t, acc):
                    dcol = (iota + t) & (D - 1)
                    a = plsc.load_gather(rows_v, [lanes, dcol])
                    b2 = plsc.load_gather(cols_v, [lanes, dcol])
                    return acc + a * b2

                acc = lax.fori_loop(0, D, t_body,
                                    jnp.zeros((L,), jnp.float32), unroll=16)
                out_v[pl.ds(eb, L)] = 1.0 / (1.0 + jnp.exp(-acc))
                return c2

            lax.fori_loop(0, C // L, group_body, 0)
            pltpu.sync_copy(out_v, out_hbm.at[pl.ds(base + ci * C, C)])

        def outer_body(i, carry):
            for b in range(NBUF):
                ci = i * NBUF + b
                compute(ci, b)

                @pl.when(ci + NBUF < n_chunks)
                def _():
                    launch(ci + NBUF, b)
            return carry

        n_main = (n_chunks // NBUF) * NBUF
        lax.fori_loop(0, n_chunks // NBUF, outer_body, 0)
        for ci in range(n_main, n_chunks):
            compute(ci, ci % NBUF)

    return body


def kernel(z, edge_index):
    ei = edge_index.astype(jnp.int32)
    return _make_sc_decoder(ei.shape[1])(z, ei[0], ei[1])
